# Initial kernel scaffold; baseline (speedup 1.0000x reference)
import jax
import jax.numpy as jnp
from jax.experimental import pallas as pl


def kernel(node_features, edge_features, edge_index, batch, W_init, b_init, W_fc, b_fc, W_node, b_node, W_edge, b_edge, W_out, b_out):
    raise NotImplementedError("write your pallas kernel here")


# speedup vs baseline: 7.5550x; 7.5550x over previous
import functools

import jax
import jax.numpy as jnp
from jax import lax
from jax.experimental import pallas as pl
from jax.experimental.pallas import tpu as pltpu
from jax.experimental.pallas import tpu_sc as plsc

N = 50000
E = 1600000
G = 64
NC = 2
NS = 16
NW = NC * NS
BLK = 128
NBLK = E // BLK
MAXI = (NBLK + NW - 1) // NW
NP = 50176
RT = NP // NS

_mesh = plsc.VectorSubcoreMesh(core_axis_name="c", subcore_axis_name="s")


@functools.partial(
    pl.kernel,
    out_type=jax.ShapeDtypeStruct((NC, NP, 8), jnp.float32),
    mesh=_mesh,
    compiler_params=pltpu.CompilerParams(use_tc_tiling_on_sc=False),
    scratch_types=[
        pltpu.VMEM((BLK,), jnp.int32),
        pltpu.VMEM((BLK, 8), jnp.float32),
        pltpu.VMEM_SHARED((NP, 8), jnp.float32),
    ],
)
def _edge_scatter(ef8_hbm, col_hbm, zeros8_hbm, out_hbm, colbuf, rows8, acc):
    cid = lax.axis_index("c")
    sid = lax.axis_index("s")
    wid = sid * NC + cid

    pltpu.sync_copy(zeros8_hbm.at[pl.ds(sid * RT, RT)],
                    acc.at[pl.ds(sid * RT, RT)])
    plsc.subcore_barrier()

    def blk_body(i, carry):
        blk = wid + NW * i

        @pl.when(blk < NBLK)
        def _():
            off = blk * BLK
            pltpu.sync_copy(col_hbm.at[pl.ds(off, BLK)], colbuf)
            pltpu.sync_copy(ef8_hbm.at[pl.ds(off, BLK)], rows8)
            pltpu.sync_copy(rows8, acc.at[colbuf], add=True)

        return carry

    lax.fori_loop(0, MAXI, blk_body, 0)

    plsc.subcore_barrier()
    pltpu.sync_copy(acc.at[pl.ds(sid * RT, RT)],
                    out_hbm.at[cid, pl.ds(sid * RT, RT)])


CH2 = NP // 8


def _node_tc_body(acc_ref, w4_ref, bn_ref, wr_ref, wc_ref, p_ref, q_ref):
    acc = acc_ref[0] + acc_ref[1]
    pre = jnp.dot(acc, w4_ref[...], preferred_element_type=jnp.float32)
    nf = jnp.maximum(pre + bn_ref[...], 0.0)
    p14 = jnp.dot(nf, wr_ref[...], preferred_element_type=jnp.float32)
    q14 = jnp.dot(nf, wc_ref[...], preferred_element_type=jnp.float32)
    z2 = jnp.zeros((CH2, 2), jnp.float32)
    p_ref[...] = jnp.concatenate([p14, z2], axis=1)
    q_ref[...] = jnp.concatenate([q14, z2], axis=1)


_node_tc = pl.pallas_call(
    _node_tc_body,
    grid=(NP // CH2,),
    in_specs=[
        pl.BlockSpec((2, CH2, 8), lambda i: (0, i, 0)),
        pl.BlockSpec((8, 10), lambda i: (0, 0)),
        pl.BlockSpec((1, 10), lambda i: (0, 0)),
        pl.BlockSpec((10, 14), lambda i: (0, 0)),
        pl.BlockSpec((10, 14), lambda i: (0, 0)),
    ],
    out_specs=[
        pl.BlockSpec((CH2, 16), lambda i: (i, 0)),
        pl.BlockSpec((CH2, 16), lambda i: (i, 0)),
    ],
    out_shape=[
        jax.ShapeDtypeStruct((NP, 16), jnp.float32),
        jax.ShapeDtypeStruct((NP, 16), jnp.float32),
    ],
)


@functools.partial(
    pl.kernel,
    out_type=jax.ShapeDtypeStruct((NC, NP, 16), jnp.float32),
    mesh=_mesh,
    compiler_params=pltpu.CompilerParams(use_tc_tiling_on_sc=False),
    scratch_types=[
        pltpu.VMEM((BLK,), jnp.int32),
        pltpu.VMEM((BLK,), jnp.int32),
        pltpu.VMEM((2 * BLK,), jnp.float32),
        pltpu.VMEM((BLK, 16), jnp.float32),
        pltpu.VMEM((BLK, 16), jnp.float32),
        pltpu.VMEM((BLK, 16), jnp.float32),
        pltpu.VMEM((4, 16), jnp.float32),
        pltpu.VMEM_SHARED((NP, 16), jnp.float32),
        pltpu.SemaphoreType.DMA,
        pltpu.SemaphoreType.DMA,
    ],
)
def _edge_gather(p_hbm, q_hbm, row_hbm, col_hbm, ef_hbm, ctab_hbm,
                 zeros16_hbm, out_hbm, rowbuf, colbuf, efbuf, pbuf, qbuf,
                 zbuf, ctab, nacc, sem1, sem2):
    cid = lax.axis_index("c")
    sid = lax.axis_index("s")
    wid = sid * NC + cid

    pltpu.sync_copy(zeros16_hbm.at[pl.ds(sid * RT, RT)],
                    nacc.at[pl.ds(sid * RT, RT)])
    pltpu.sync_copy(ctab_hbm, ctab)
    a0 = ctab[0]
    a1 = ctab[1]
    ab = ctab[2]
    plsc.subcore_barrier()

    def blk_body(i, carry):
        blk = wid + NW * i

        @pl.when(blk < NBLK)
        def _():
            off = blk * BLK
            pltpu.sync_copy(row_hbm.at[pl.ds(off, BLK)], rowbuf)
            pltpu.sync_copy(col_hbm.at[pl.ds(off, BLK)], colbuf)
            pltpu.sync_copy(ef_hbm.at[pl.ds(2 * off, 2 * BLK)], efbuf)
            cp = pltpu.async_copy(p_hbm.at[rowbuf], pbuf, sem1)
            cq = pltpu.async_copy(q_hbm.at[colbuf], qbuf, sem2)
            cp.wait()
            cq.wait()

            for j in range(BLK // 8):
                ev = efbuf[pl.ds(16 * j, 16)]
                for k in range(8):
                    m = 8 * j + k
                    v = (pbuf[m] + qbuf[m]
                         + ev[2 * k] * a0 + ev[2 * k + 1] * a1 + ab)
                    zbuf[m] = jnp.maximum(v, 0.0)

            pltpu.sync_copy(zbuf, nacc.at[rowbuf], add=True)

        return carry

    lax.fori_loop(0, MAXI, blk_body, 0)

    plsc.subcore_barrier()
    pltpu.sync_copy(nacc.at[pl.ds(sid * RT, RT)],
                    out_hbm.at[cid, pl.ds(sid * RT, RT)])


CH4 = NP // 16


def _pool_tc_body(nacc_ref, batch_ref, s_ref):
    z = nacc_ref[0] + nacc_ref[1]
    b = batch_ref[0]
    gids = lax.broadcasted_iota(jnp.int32, (G, CH4), 0)
    oh = (gids == b).astype(jnp.float32)
    part = jnp.dot(oh, z, preferred_element_type=jnp.float32)

    @pl.when(pl.program_id(0) == 0)
    def _():
        s_ref[...] = jnp.zeros((G, 16), jnp.float32)

    s_ref[...] += part


_pool_tc = pl.pallas_call(
    _pool_tc_body,
    grid=(NP // CH4,),
    in_specs=[
        pl.BlockSpec((2, CH4, 16), lambda i: (0, i, 0)),
        pl.BlockSpec((1, 1, CH4), lambda i: (i, 0, 0)),
    ],
    out_specs=pl.BlockSpec((G, 16), lambda i: (0, 0)),
    out_shape=jax.ShapeDtypeStruct((G, 16), jnp.float32),
)


def kernel(node_features, edge_features, edge_index, batch, W_init, b_init,
           W_fc, b_fc, W_node, b_node, W_edge, b_edge, W_out, b_out):
    row = edge_index[0].astype(jnp.int32)
    col = edge_index[1].astype(jnp.int32)

    nf0 = W_init[:, 0] + b_init
    Wn_a = W_node[:, :2]
    Wn_b = W_node[:, 2:]
    bn = Wn_a @ nf0 + b_node
    M2 = Wn_b @ W_fc
    mb = Wn_b @ b_fc
    W8 = jnp.concatenate([M2, mb[:, None], jnp.zeros((10, 5))], 1).T
    We_e = W_edge[:, :13]
    We_r = W_edge[:, 13:23]
    We_c = W_edge[:, 23:]
    A2 = We_e @ W_fc
    ab = We_e @ b_fc + b_edge
    pad = jnp.zeros((2,), jnp.float32)
    A0 = jnp.concatenate([A2[:, 0], pad])
    A1 = jnp.concatenate([A2[:, 1], pad])
    AB = jnp.concatenate([ab, jnp.array([1.0, 0.0], jnp.float32)])
    ctab = jnp.stack([A0, A1, AB, jnp.zeros((16,), jnp.float32)])

    ef8 = jnp.concatenate(
        [edge_features,
         jnp.ones((E, 1), jnp.float32),
         jnp.zeros((E, 5), jnp.float32)], axis=1)

    zeros8 = jnp.zeros((NP, 8), jnp.float32)
    zeros16 = jnp.zeros((NP, 16), jnp.float32)
    batch_f = jnp.pad(batch.astype(jnp.int32), (0, NP - N),
                      constant_values=-1).reshape(16, 1, CH4)

    acc2 = _edge_scatter(ef8, col, zeros8)

    P, Q = _node_tc(acc2, W8, bn.reshape(1, 10), We_r.T, We_c.T)

    ef_flat = edge_features.reshape(-1)
    Z2 = _edge_gather(P, Q, row, col, ef_flat, ctab, zeros16)

    S = _pool_tc(Z2, batch_f)

    cnt = S[:, 14]
    raw = S[:, :14] @ W_out[0] + b_out[0] * cnt
    logits = raw / jnp.maximum(cnt, 1.0)
    return jax.nn.sigmoid(logits)
